# baseline (speedup 1.0000x reference)
import jax
import jax.numpy as jnp
from jax.experimental import pallas as pl

N = 10000
K = 16
IN_FEATS = 256
OUT_FEATS = 256

BN = 400


def _fused_kernel(x_ref, wt_ref, b_ref, o_ref):
    x = x_ref[...].reshape(BN * K, IN_FEATS).astype(jnp.bfloat16)
    y = jnp.dot(x, wt_ref[...].astype(jnp.bfloat16),
                preferred_element_type=jnp.float32)
    m = jnp.max(y.reshape(BN, K, OUT_FEATS), axis=1)
    o_ref[...] = m + b_ref[...]


@jax.jit
def kernel(neighbour, W, b):
    wt = W.T
    b2 = b.reshape(1, OUT_FEATS)
    grid = (N // BN,)
    return pl.pallas_call(
        _fused_kernel,
        grid=grid,
        in_specs=[
            pl.BlockSpec((BN, K, IN_FEATS), lambda i: (i, 0, 0)),
            pl.BlockSpec((IN_FEATS, OUT_FEATS), lambda i: (0, 0)),
            pl.BlockSpec((1, OUT_FEATS), lambda i: (0, 0)),
        ],
        out_specs=pl.BlockSpec((BN, OUT_FEATS), lambda i: (i, 0)),
        out_shape=jax.ShapeDtypeStruct((N, OUT_FEATS), jnp.float32),
    )(neighbour, wt, b2)


# speedup vs baseline: 1.1044x; 1.1044x over previous
import jax
import jax.numpy as jnp
from jax.experimental import pallas as pl

N = 10000
K = 16
IN_FEATS = 256
OUT_FEATS = 256

BN = 1000


def _fused_kernel(x_ref, wt_ref, b_ref, o_ref):
    x = x_ref[...].reshape(BN * K, IN_FEATS)
    y = jnp.dot(x, wt_ref[...], preferred_element_type=jnp.float32)
    m = jnp.max(y.reshape(BN, K, OUT_FEATS), axis=1)
    o_ref[...] = m + b_ref[...]


@jax.jit
def kernel(neighbour, W, b):
    wt = W.T
    b2 = b.reshape(1, OUT_FEATS)
    grid = (N // BN,)
    return pl.pallas_call(
        _fused_kernel,
        grid=grid,
        in_specs=[
            pl.BlockSpec((BN, K, IN_FEATS), lambda i: (i, 0, 0)),
            pl.BlockSpec((IN_FEATS, OUT_FEATS), lambda i: (0, 0)),
            pl.BlockSpec((1, OUT_FEATS), lambda i: (0, 0)),
        ],
        out_specs=pl.BlockSpec((BN, OUT_FEATS), lambda i: (i, 0)),
        out_shape=jax.ShapeDtypeStruct((N, OUT_FEATS), jnp.float32),
    )(neighbour, wt, b2)
